# initial kernel scaffold (unmeasured)
import math

import jax
import jax.numpy as jnp
from jax import lax
from jax.experimental import pallas as pl
from jax.experimental.pallas import tpu as pltpu

N_DEV = 4
B = 32
NB = 256
BS = 32
H = 16
D = 128


def _body(w_ref, n_ref, q_ref, k_hbm, v_hbm, out_ref,
          kbuf, vbuf, ksem, vsem,
          l_scr, o_scr, comm_o, comm_l,
          o_send, o_recv, l_send, l_recv):
    pages = k_hbm.shape[0]
    my = lax.axis_index("i")
    left = lax.rem(my + N_DEV - 1, N_DEV)
    right = lax.rem(my + 1, N_DEV)

    barrier = pltpu.get_barrier_semaphore()
    for nbr in (left, right):
        pl.semaphore_signal(barrier, inc=1, device_id=(nbr,),
                            device_id_type=pl.DeviceIdType.MESH)
    pl.semaphore_wait(barrier, 2)

    l_scr[...] = jnp.zeros_like(l_scr)
    o_scr[...] = jnp.zeros_like(o_scr)

    n = n_ref[0]
    scale = 1.0 / math.sqrt(D)

    def fetch(item, slot):
        page = w_ref[item] % pages
        pltpu.make_async_copy(k_hbm.at[page], kbuf.at[slot],
                              ksem.at[slot]).start()
        pltpu.make_async_copy(v_hbm.at[page], vbuf.at[slot],
                              vsem.at[slot]).start()

    @pl.when(n > 0)
    def _():
        fetch(0, 0)

    def step(it, carry):
        slot = lax.rem(it, 2)

        @pl.when(it + 1 < n)
        def _():
            fetch(it + 1, 1 - slot)

        w = w_ref[it]
        b = w // pages
        page = w % pages
        pltpu.make_async_copy(k_hbm.at[page], kbuf.at[slot],
                              ksem.at[slot]).wait()
        pltpu.make_async_copy(v_hbm.at[page], vbuf.at[slot],
                              vsem.at[slot]).wait()

        q = q_ref[b, 0]
        k = kbuf[slot]
        v = vbuf[slot]
        s = jnp.sum(q[None, :, :] * k, axis=-1, keepdims=True) * scale
        p = jnp.exp(s)
        l_scr[b] = l_scr[b] + jnp.sum(p, axis=0)
        o_scr[b] = o_scr[b] + jnp.sum(p * v, axis=0)
        return carry

    lax.fori_loop(0, n, step, 0)

    comm_o[0] = o_scr[...]
    comm_l[0] = l_scr[...]
    for h in range(N_DEV - 1):
        snd = h % 2
        rcv = (h + 1) % 2
        ro = pltpu.make_async_remote_copy(
            src_ref=comm_o.at[snd], dst_ref=comm_o.at[rcv],
            send_sem=o_send.at[snd], recv_sem=o_recv.at[rcv],
            device_id=(right,), device_id_type=pl.DeviceIdType.MESH)
        rl = pltpu.make_async_remote_copy(
            src_ref=comm_l.at[snd], dst_ref=comm_l.at[rcv],
            send_sem=l_send.at[snd], recv_sem=l_recv.at[rcv],
            device_id=(right,), device_id_type=pl.DeviceIdType.MESH)
        ro.start()
        rl.start()
        ro.wait()
        rl.wait()
        o_scr[...] = o_scr[...] + comm_o[rcv]
        l_scr[...] = l_scr[...] + comm_l[rcv]

    out_ref[:, 0] = o_scr[...] / l_scr[...]


def kernel(Q, K, V, bt, lens):
    pages = K.shape[0]
    rank = lax.axis_index("i")

    slot_idx = jnp.arange(NB, dtype=jnp.int32)[None, :]
    valid = (slot_idx < lens[:, None]) & ((bt // pages) == rank)
    wvals = (jnp.arange(B, dtype=jnp.int32)[:, None] * pages
             + (bt % pages)).astype(jnp.int32)
    flat_v = valid.reshape(-1)
    flat_w = jnp.where(flat_v, wvals.reshape(-1), 0)
    idx = jnp.arange(B * NB, dtype=jnp.int32)
    keys = jnp.where(flat_v, idx, jnp.int32(B * NB))
    _, wlist = lax.sort((keys, flat_w), num_keys=1)
    n = jnp.sum(flat_v).astype(jnp.int32).reshape(1)

    return pl.pallas_call(
        _body,
        out_shape=jax.ShapeDtypeStruct((B, 1, H, D), jnp.float32),
        in_specs=[
            pl.BlockSpec(memory_space=pltpu.SMEM),
            pl.BlockSpec(memory_space=pltpu.SMEM),
            pl.BlockSpec(memory_space=pltpu.VMEM),
            pl.BlockSpec(memory_space=pl.ANY),
            pl.BlockSpec(memory_space=pl.ANY),
        ],
        out_specs=pl.BlockSpec(memory_space=pltpu.VMEM),
        scratch_shapes=[
            pltpu.VMEM((2, BS, H, D), jnp.float32),
            pltpu.VMEM((2, BS, H, D), jnp.float32),
            pltpu.SemaphoreType.DMA((2,)),
            pltpu.SemaphoreType.DMA((2,)),
            pltpu.VMEM((B, H, 1), jnp.float32),
            pltpu.VMEM((B, H, D), jnp.float32),
            pltpu.VMEM((2, B, H, D), jnp.float32),
            pltpu.VMEM((2, B, H, 1), jnp.float32),
            pltpu.SemaphoreType.DMA((2,)),
            pltpu.SemaphoreType.DMA((2,)),
            pltpu.SemaphoreType.DMA((2,)),
            pltpu.SemaphoreType.DMA((2,)),
        ],
        compiler_params=pltpu.CompilerParams(collective_id=0),
    )(wlist, n, Q, K, V)


# baseline (device time: 794568 ns/iter reference)
import math

import jax
import jax.numpy as jnp
from jax import lax
from jax.experimental import pallas as pl
from jax.experimental.pallas import tpu as pltpu

N_DEV = 4
B = 32
NB = 256
BS = 32
H = 16
D = 128


def _body(w_ref, n_ref, q_ref, k_hbm, v_hbm, out_ref,
          kbuf, vbuf, ksem, vsem,
          l_scr, o_scr, comm_o, comm_l,
          o_send, o_recv, l_send, l_recv):
    pages = k_hbm.shape[0]
    my = lax.axis_index("i")
    left = lax.rem(my + N_DEV - 1, N_DEV)
    right = lax.rem(my + 1, N_DEV)

    barrier = pltpu.get_barrier_semaphore()
    for nbr in (left, right):
        pl.semaphore_signal(barrier, inc=1, device_id=(nbr,),
                            device_id_type=pl.DeviceIdType.MESH)
    pl.semaphore_wait(barrier, 2)

    l_scr[...] = jnp.zeros_like(l_scr)
    o_scr[...] = jnp.zeros_like(o_scr)

    n = n_ref[0]
    scale = 1.0 / math.sqrt(D)

    def fetch(item, slot):
        page = w_ref[item] % pages
        pltpu.make_async_copy(k_hbm.at[page], kbuf.at[slot],
                              ksem.at[slot]).start()
        pltpu.make_async_copy(v_hbm.at[page], vbuf.at[slot],
                              vsem.at[slot]).start()

    @pl.when(n > 0)
    def _():
        fetch(0, 0)

    def step(it, carry):
        slot = lax.rem(it, 2)

        @pl.when(it + 1 < n)
        def _():
            fetch(it + 1, 1 - slot)

        w = w_ref[it]
        b = w // pages
        page = w % pages
        pltpu.make_async_copy(k_hbm.at[page], kbuf.at[slot],
                              ksem.at[slot]).wait()
        pltpu.make_async_copy(v_hbm.at[page], vbuf.at[slot],
                              vsem.at[slot]).wait()

        q = q_ref[b, 0]
        k = kbuf[slot]
        v = vbuf[slot]
        s = jnp.sum(q[None, :, :] * k, axis=-1, keepdims=True) * scale
        p = jnp.exp(s)
        l_scr[b] = l_scr[b] + jnp.sum(p, axis=0)
        o_scr[b] = o_scr[b] + jnp.sum(p * v, axis=0)
        return carry

    lax.fori_loop(0, n, step, 0)

    comm_o[0] = o_scr[...]
    comm_l[0] = l_scr[...]
    for h in range(N_DEV - 1):
        ro = pltpu.make_async_remote_copy(
            src_ref=comm_o.at[h], dst_ref=comm_o.at[h + 1],
            send_sem=o_send.at[h], recv_sem=o_recv.at[h],
            device_id=(right,), device_id_type=pl.DeviceIdType.MESH)
        rl = pltpu.make_async_remote_copy(
            src_ref=comm_l.at[h], dst_ref=comm_l.at[h + 1],
            send_sem=l_send.at[h], recv_sem=l_recv.at[h],
            device_id=(right,), device_id_type=pl.DeviceIdType.MESH)
        ro.start()
        rl.start()
        ro.wait()
        rl.wait()
        o_scr[...] = o_scr[...] + comm_o[h + 1]
        l_scr[...] = l_scr[...] + comm_l[h + 1]

    out_ref[:, 0] = o_scr[...] / l_scr[...]


def kernel(Q, K, V, bt, lens):
    pages = K.shape[0]
    rank = lax.axis_index("i")

    slot_idx = jnp.arange(NB, dtype=jnp.int32)[None, :]
    valid = (slot_idx < lens[:, None]) & ((bt // pages) == rank)
    wvals = (jnp.arange(B, dtype=jnp.int32)[:, None] * pages
             + (bt % pages)).astype(jnp.int32)
    flat_v = valid.reshape(-1)
    flat_w = jnp.where(flat_v, wvals.reshape(-1), 0)
    idx = jnp.arange(B * NB, dtype=jnp.int32)
    keys = jnp.where(flat_v, idx, jnp.int32(B * NB))
    _, wlist = lax.sort((keys, flat_w), num_keys=1)
    n = jnp.sum(flat_v).astype(jnp.int32).reshape(1)

    return pl.pallas_call(
        _body,
        out_shape=jax.ShapeDtypeStruct((B, 1, H, D), jnp.float32),
        in_specs=[
            pl.BlockSpec(memory_space=pltpu.SMEM),
            pl.BlockSpec(memory_space=pltpu.SMEM),
            pl.BlockSpec(memory_space=pltpu.VMEM),
            pl.BlockSpec(memory_space=pl.ANY),
            pl.BlockSpec(memory_space=pl.ANY),
        ],
        out_specs=pl.BlockSpec(memory_space=pltpu.VMEM),
        scratch_shapes=[
            pltpu.VMEM((2, BS, H, D), jnp.float32),
            pltpu.VMEM((2, BS, H, D), jnp.float32),
            pltpu.SemaphoreType.DMA((2,)),
            pltpu.SemaphoreType.DMA((2,)),
            pltpu.VMEM((B, H, 1), jnp.float32),
            pltpu.VMEM((B, H, D), jnp.float32),
            pltpu.VMEM((N_DEV, B, H, D), jnp.float32),
            pltpu.VMEM((N_DEV, B, H, 1), jnp.float32),
            pltpu.SemaphoreType.DMA((N_DEV - 1,)),
            pltpu.SemaphoreType.DMA((N_DEV - 1,)),
            pltpu.SemaphoreType.DMA((N_DEV - 1,)),
            pltpu.SemaphoreType.DMA((N_DEV - 1,)),
        ],
        compiler_params=pltpu.CompilerParams(collective_id=0),
    )(wlist, n, Q, K, V)
